# Initial kernel scaffold; baseline (speedup 1.0000x reference)
import jax
import jax.numpy as jnp
from jax.experimental import pallas as pl


def kernel(mention_hidden_state, entity_hidden_state, sent_hidden_state, virtual_hidden_state, edge_index, type_emb, ln_gamma, ln_beta, W_self, W_nbr, b):
    raise NotImplementedError("write your pallas kernel here")


# speedup vs baseline: 4.5079x; 4.5079x over previous
import functools

import jax
import jax.numpy as jnp
from jax import lax
from jax.experimental import pallas as pl
from jax.experimental.pallas import tpu as pltpu, tpu_sc as plsc

B, NM, NE, NS, NV = 4, 1200, 600, 500, 200
H, TE, D = 96, 32, 128
N = NM + NE + NS + NV
BN = B * N
EDGES = 320000

NC, NSUB = 2, 16
NW = NC * NSUB
CHUNK = 128
CHUNKS_PER_W = 79
EP = NW * CHUNKS_PER_W * CHUNK
NCH = EP // CHUNK

ACC_ROWS = 2432
DUMMY = 2400
ROWS_PER_TILE = ACC_ROWS // NSUB
DEGW = 128


def _ln_body(f_ref, t_ref, g_ref, b_ref, x_ref):
    x = jnp.concatenate([f_ref[...], t_ref[...]], axis=1)
    mu = jnp.mean(x, axis=1, keepdims=True)
    var = jnp.mean((x - mu) * (x - mu), axis=1, keepdims=True)
    xn = (x - mu) * lax.rsqrt(var + 1e-5)
    x_ref[...] = xn * g_ref[...] + b_ref[...]


def _ln_concat(feats, trows, gamma, beta):
    blk = 1000
    grid = BN // blk
    return pl.pallas_call(
        _ln_body,
        grid=(grid,),
        in_specs=[
            pl.BlockSpec((blk, H), lambda i: (i, 0)),
            pl.BlockSpec((blk, TE), lambda i: (i, 0)),
            pl.BlockSpec((1, D), lambda i: (0, 0)),
            pl.BlockSpec((1, D), lambda i: (0, 0)),
        ],
        out_specs=pl.BlockSpec((blk, D), lambda i: (i, 0)),
        out_shape=jax.ShapeDtypeStruct((BN, D), jnp.float32),
    )(feats, trows, gamma, beta)


def _i16(v):
    return jnp.full((16,), v, jnp.int32)


def _dst_to_local(d16):
    one = _i16(1)
    zero = _i16(0)
    bidx = (jnp.where(d16 >= _i16(N), one, zero)
            + jnp.where(d16 >= _i16(2 * N), one, zero)
            + jnp.where(d16 >= _i16(3 * N), one, zero))
    r = d16 - bidx * _i16(N)
    ent = jnp.logical_and(r >= _i16(NM), r < _i16(NM + NE))
    return jnp.where(ent, bidx * _i16(NE) + (r - _i16(NM)), _i16(DUMMY))


def _sc_body(edge_hbm, x_hbm, agg_out, deg_out,
             edge_v, loc_v, rows_v, ones_v, zrow,
             acc_sh, deg_sh, sem):
    c = lax.axis_index("c")
    s = lax.axis_index("s")
    w = c * NSUB + s

    zeros16 = jnp.zeros((16,), jnp.float32)
    ones16 = jnp.ones((16,), jnp.float32)
    for r in range(8):
        for i in range(8):
            zrow[r, pl.ds(16 * i, 16)] = zeros16
    for r in range(CHUNK):
        for i in range(8):
            ones_v[r, pl.ds(16 * i, 16)] = ones16

    base = s * ROWS_PER_TILE
    for k in range(ROWS_PER_TILE // 8):
        pltpu.sync_copy(zrow, acc_sh.at[pl.ds(base + 8 * k, 8)])
        pltpu.sync_copy(zrow, deg_sh.at[pl.ds(base + 8 * k, 8)])
    plsc.subcore_barrier()

    def chunk_body(j, carry):
        ci = w * CHUNKS_PER_W + j
        pltpu.sync_copy(edge_hbm.at[:, pl.ds(ci, 1), :], edge_v)
        for i in range(CHUNK // 16):
            d16 = edge_v[1, 0, pl.ds(16 * i, 16)]
            loc = _dst_to_local(d16)
            loc_v[pl.ds(16 * i, 16)] = loc
        cp = pltpu.async_copy(x_hbm.at[edge_v.at[0, 0]], rows_v, sem)
        cp.wait()
        pltpu.sync_copy(rows_v, acc_sh.at[loc_v], add=True)
        pltpu.sync_copy(ones_v, deg_sh.at[loc_v], add=True)
        return carry

    lax.fori_loop(0, CHUNKS_PER_W, chunk_body, 0)
    plsc.subcore_barrier()

    pltpu.sync_copy(acc_sh.at[pl.ds(base, ROWS_PER_TILE)],
                    agg_out.at[c, pl.ds(base, ROWS_PER_TILE)])
    pltpu.sync_copy(deg_sh.at[pl.ds(base, ROWS_PER_TILE)],
                    deg_out.at[c, pl.ds(base, ROWS_PER_TILE)])


def _sc_scatter(edges3, x):
    mesh = plsc.VectorSubcoreMesh(core_axis_name="c", subcore_axis_name="s")
    fn = pl.kernel(
        _sc_body,
        out_type=(
            jax.ShapeDtypeStruct((NC, ACC_ROWS, D), jnp.float32),
            jax.ShapeDtypeStruct((NC, ACC_ROWS, DEGW), jnp.float32),
        ),
        mesh=mesh,
        scratch_types=[
            pltpu.VMEM((2, 1, CHUNK), jnp.int32),
            pltpu.VMEM((CHUNK,), jnp.int32),
            pltpu.VMEM((CHUNK, D), jnp.float32),
            pltpu.VMEM((CHUNK, DEGW), jnp.float32),
            pltpu.VMEM((8, D), jnp.float32),
            pltpu.VMEM_SHARED((ACC_ROWS, D), jnp.float32),
            pltpu.VMEM_SHARED((ACC_ROWS, DEGW), jnp.float32),
            pltpu.SemaphoreType.DMA,
        ],
    )
    return fn(edges3, x)


def _combine_body(xe_ref, agg_ref, deg_ref, ws_ref, wn_ref, b_ref, o_ref):
    xe = xe_ref[...]
    selfm = jnp.dot(xe, ws_ref[...], preferred_element_type=jnp.float32)
    acc = agg_ref[0] + agg_ref[1]
    aggm = jnp.dot(acc[:B * NE], wn_ref[...], preferred_element_type=jnp.float32)
    deg = deg_ref[0, :, 0:1] + deg_ref[1, :, 0:1]
    d = jnp.maximum(deg[:B * NE], 1.0)
    o_ref[...] = jnp.maximum(selfm + aggm / d + b_ref[...], 0.0)


def _combine(xe, agg, deg, w_self, w_nbr, bias):
    return pl.pallas_call(
        _combine_body,
        out_shape=jax.ShapeDtypeStruct((B * NE, D), jnp.float32),
    )(xe, agg, deg, w_self, w_nbr, bias)


def kernel(mention_hidden_state, entity_hidden_state, sent_hidden_state,
           virtual_hidden_state, edge_index, type_emb, ln_gamma, ln_beta,
           W_self, W_nbr, b):
    feats = jnp.concatenate(
        [mention_hidden_state, entity_hidden_state,
         sent_hidden_state, virtual_hidden_state], axis=1).reshape(BN, H)
    tnode = jnp.concatenate([
        jnp.broadcast_to(type_emb[0], (NM, TE)),
        jnp.broadcast_to(type_emb[1], (NE, TE)),
        jnp.broadcast_to(type_emb[2], (NS, TE)),
        jnp.broadcast_to(type_emb[3], (NV, TE)),
    ], axis=0)
    trows = jnp.broadcast_to(tnode[None], (B, N, TE)).reshape(BN, TE)

    x = _ln_concat(feats, trows, ln_gamma.reshape(1, D), ln_beta.reshape(1, D))

    edges3 = jnp.pad(edge_index, ((0, 0), (0, EP - EDGES))).reshape(2, NCH, CHUNK)
    agg, deg = _sc_scatter(edges3, x)

    xe = x.reshape(B, N, D)[:, NM:NM + NE].reshape(B * NE, D)
    out = _combine(xe, agg, deg, W_self, W_nbr, b.reshape(1, D))
    return out.reshape(B, NE, D)


# speedup vs baseline: 8.9043x; 1.9753x over previous
import functools

import jax
import jax.numpy as jnp
from jax import lax
from jax.experimental import pallas as pl
from jax.experimental.pallas import tpu as pltpu, tpu_sc as plsc

B, NM, NE, NS, NV = 4, 1200, 600, 500, 200
H, TE, D = 96, 32, 128
N = NM + NE + NS + NV
BN = B * N
EDGES = 320000

NC, NSUB = 2, 16
NW = NC * NSUB
CHUNK = 128
CHUNKS_PER_W = 79
EP = NW * CHUNKS_PER_W * CHUNK
NCH = EP // CHUNK
EDGES_PER_W = CHUNKS_PER_W * CHUNK
PCAP = EDGES_PER_W + CHUNK

ACC_ROWS = 2432
DUMMY = 2400
ROWS_PER_TILE = ACC_ROWS // NSUB


def _ln_body(f_ref, t_ref, g_ref, b_ref, x_ref):
    x = jnp.concatenate([f_ref[...], t_ref[...]], axis=1)
    mu = jnp.mean(x, axis=1, keepdims=True)
    var = jnp.mean((x - mu) * (x - mu), axis=1, keepdims=True)
    xn = (x - mu) * lax.rsqrt(var + 1e-5)
    x_ref[...] = xn * g_ref[...] + b_ref[...]


def _ln_concat(feats, trows, gamma, beta):
    blk = 1000
    grid = BN // blk
    return pl.pallas_call(
        _ln_body,
        grid=(grid,),
        in_specs=[
            pl.BlockSpec((blk, H), lambda i: (i, 0)),
            pl.BlockSpec((blk, TE), lambda i: (i, 0)),
            pl.BlockSpec((1, D), lambda i: (0, 0)),
            pl.BlockSpec((1, D), lambda i: (0, 0)),
        ],
        out_specs=pl.BlockSpec((blk, D), lambda i: (i, 0)),
        out_shape=jax.ShapeDtypeStruct((BN, D), jnp.float32),
    )(feats, trows, gamma, beta)


def _i16(v):
    return jnp.full((16,), v, jnp.int32)


def _dst_to_local(d16):
    one = _i16(1)
    zero = _i16(0)
    bidx = (jnp.where(d16 >= _i16(N), one, zero)
            + jnp.where(d16 >= _i16(2 * N), one, zero)
            + jnp.where(d16 >= _i16(3 * N), one, zero))
    r = d16 - bidx * _i16(N)
    ent = jnp.logical_and(r >= _i16(NM), r < _i16(NM + NE))
    loc = jnp.where(ent, bidx * _i16(NE) + (r - _i16(NM)), _i16(DUMMY))
    return loc, ent


def _sc_body(edge_hbm, x_hbm, agg_out, deg_out,
             edge_v, locbuf, rows_v, ones_v, zrow, psrc, ploc,
             acc_sh, deg_sh, stage_sh, sem):
    c = lax.axis_index("c")
    s = lax.axis_index("s")
    w = c * NSUB + s

    zeros16 = jnp.zeros((16,), jnp.float32)
    ones16 = jnp.ones((16,), jnp.float32)
    for r in range(8):
        for i in range(8):
            zrow[r, pl.ds(16 * i, 16)] = zeros16
    for r in range(CHUNK):
        for i in range(8):
            ones_v[r, pl.ds(16 * i, 16)] = ones16

    base = s * ROWS_PER_TILE
    for k in range(ROWS_PER_TILE // 8):
        pltpu.sync_copy(zrow, acc_sh.at[pl.ds(base + 8 * k, 8)])
        pltpu.sync_copy(zrow, deg_sh.at[pl.ds(base + 8 * k, 8)])
    plsc.subcore_barrier()

    def scan_body(j, cnt_vec):
        ci = w * CHUNKS_PER_W + j
        pltpu.sync_copy(edge_hbm.at[:, pl.ds(ci, 1), :], edge_v)
        for i in range(CHUNK // 16):
            s16 = edge_v[0, 0, pl.ds(16 * i, 16)]
            d16 = edge_v[1, 0, pl.ds(16 * i, 16)]
            loc, ent = _dst_to_local(d16)
            pos = cnt_vec + jnp.cumsum(
                jnp.where(ent, _i16(1), _i16(0))) - _i16(1)
            plsc.store_scatter(psrc, [pos], s16, mask=ent)
            plsc.store_scatter(ploc, [pos], loc, mask=ent)
            cnt_vec = cnt_vec + plsc.all_reduce_population_count(ent)
        return cnt_vec

    cnt_vec = lax.fori_loop(0, CHUNKS_PER_W, scan_body, _i16(0))

    iota16 = lax.iota(jnp.int32, 16)
    for k in range(CHUNK // 16):
        tail = cnt_vec + iota16 + _i16(16 * k)
        plsc.store_scatter(psrc, [tail], _i16(0))
        plsc.store_scatter(ploc, [tail], _i16(DUMMY))
    cnt = jnp.max(cnt_vec)
    nch = (cnt + (CHUNK - 1)) // CHUNK
    plsc.subcore_barrier()

    def chunk_body(j, carry):
        off = j * CHUNK
        cp = pltpu.async_copy(x_hbm.at[psrc.at[pl.ds(off, CHUNK)]], rows_v, sem)
        cp.wait()
        pltpu.sync_copy(rows_v, acc_sh.at[ploc.at[pl.ds(off, CHUNK)]], add=True)
        pltpu.sync_copy(ones_v, deg_sh.at[ploc.at[pl.ds(off, CHUNK)]], add=True)
        return carry

    lax.fori_loop(0, nch, chunk_body, 0)
    plsc.subcore_barrier()

    pltpu.sync_copy(acc_sh.at[pl.ds(base, ROWS_PER_TILE)],
                    agg_out.at[c, pl.ds(base, ROWS_PER_TILE)])
    pltpu.sync_copy(deg_sh.at[pl.ds(base, ROWS_PER_TILE)],
                    deg_out.at[c, pl.ds(base, ROWS_PER_TILE)])


def _sc_scatter(edges3, x):
    mesh = plsc.VectorSubcoreMesh(core_axis_name="c", subcore_axis_name="s")
    fn = pl.kernel(
        _sc_body,
        out_type=(
            jax.ShapeDtypeStruct((NC, ACC_ROWS, D), jnp.float32),
            jax.ShapeDtypeStruct((NC, ACC_ROWS, D), jnp.float32),
        ),
        mesh=mesh,
        compiler_params=pltpu.CompilerParams(needs_layout_passes=False),
        scratch_types=[
            pltpu.VMEM((2, 1, CHUNK), jnp.int32),
            pltpu.VMEM((CHUNK,), jnp.int32),
            pltpu.VMEM((CHUNK, D), jnp.float32),
            pltpu.VMEM((CHUNK, D), jnp.float32),
            pltpu.VMEM((8, D), jnp.float32),
            pltpu.VMEM((PCAP,), jnp.int32),
            pltpu.VMEM((PCAP,), jnp.int32),
            pltpu.VMEM_SHARED((ACC_ROWS, D), jnp.float32),
            pltpu.VMEM_SHARED((ACC_ROWS, D), jnp.float32),
            pltpu.VMEM_SHARED((NSUB, CHUNK), jnp.int32),
            pltpu.SemaphoreType.DMA,
        ],
    )
    return fn(edges3, x)


def _combine_body(xe_ref, agg_ref, deg_ref, ws_ref, wn_ref, b_ref, o_ref):
    xe = xe_ref[...]
    selfm = jnp.dot(xe, ws_ref[...], preferred_element_type=jnp.float32)
    acc = agg_ref[0] + agg_ref[1]
    aggm = jnp.dot(acc[:B * NE], wn_ref[...], preferred_element_type=jnp.float32)
    deg = deg_ref[0, :, 0:1] + deg_ref[1, :, 0:1]
    d = jnp.maximum(deg[:B * NE], 1.0)
    o_ref[...] = jnp.maximum(selfm + aggm / d + b_ref[...], 0.0)


def _combine(xe, agg, deg, w_self, w_nbr, bias):
    return pl.pallas_call(
        _combine_body,
        out_shape=jax.ShapeDtypeStruct((B * NE, D), jnp.float32),
    )(xe, agg, deg, w_self, w_nbr, bias)


def kernel(mention_hidden_state, entity_hidden_state, sent_hidden_state,
           virtual_hidden_state, edge_index, type_emb, ln_gamma, ln_beta,
           W_self, W_nbr, b):
    feats = jnp.concatenate(
        [mention_hidden_state, entity_hidden_state,
         sent_hidden_state, virtual_hidden_state], axis=1).reshape(BN, H)
    tnode = jnp.concatenate([
        jnp.broadcast_to(type_emb[0], (NM, TE)),
        jnp.broadcast_to(type_emb[1], (NE, TE)),
        jnp.broadcast_to(type_emb[2], (NS, TE)),
        jnp.broadcast_to(type_emb[3], (NV, TE)),
    ], axis=0)
    trows = jnp.broadcast_to(tnode[None], (B, N, TE)).reshape(BN, TE)

    x = _ln_concat(feats, trows, ln_gamma.reshape(1, D), ln_beta.reshape(1, D))

    edges3 = jnp.pad(edge_index, ((0, 0), (0, EP - EDGES))).reshape(2, NCH, CHUNK)
    agg, deg = _sc_scatter(edges3, x)

    xe = x.reshape(B, N, D)[:, NM:NM + NE].reshape(B * NE, D)
    out = _combine(xe, agg, deg, W_self, W_nbr, b.reshape(1, D))
    return out.reshape(B, NE, D)


# speedup vs baseline: 11.0948x; 1.2460x over previous
import functools

import jax
import jax.numpy as jnp
from jax import lax
from jax.experimental import pallas as pl
from jax.experimental.pallas import tpu as pltpu, tpu_sc as plsc

B, NM, NE, NS, NV = 4, 1200, 600, 500, 200
H, TE, D = 96, 32, 128
N = NM + NE + NS + NV
BN = B * N
EDGES = 320000

NC, NSUB = 2, 16
NW = NC * NSUB
CHUNK = 128
EBLK = 8
NLB = 10
CHUNKS_PER_W = EBLK * NLB
EP = NW * CHUNKS_PER_W * CHUNK
NCH = EP // CHUNK
EDGES_PER_W = CHUNKS_PER_W * CHUNK
PCAP = EDGES_PER_W + CHUNK

ACC_ROWS = 2432
DUMMY = 2400
ROWS_PER_TILE = ACC_ROWS // NSUB


def _ln_body(f_ref, t_ref, g_ref, b_ref, x_ref):
    x = jnp.concatenate([f_ref[...], t_ref[...]], axis=1)
    mu = jnp.mean(x, axis=1, keepdims=True)
    var = jnp.mean((x - mu) * (x - mu), axis=1, keepdims=True)
    xn = (x - mu) * lax.rsqrt(var + 1e-5)
    x_ref[...] = xn * g_ref[...] + b_ref[...]


def _ln_concat(feats, trows, gamma, beta):
    blk = 1000
    grid = BN // blk
    return pl.pallas_call(
        _ln_body,
        grid=(grid,),
        in_specs=[
            pl.BlockSpec((blk, H), lambda i: (i, 0)),
            pl.BlockSpec((blk, TE), lambda i: (i, 0)),
            pl.BlockSpec((1, D), lambda i: (0, 0)),
            pl.BlockSpec((1, D), lambda i: (0, 0)),
        ],
        out_specs=pl.BlockSpec((blk, D), lambda i: (i, 0)),
        out_shape=jax.ShapeDtypeStruct((BN, D), jnp.float32),
    )(feats, trows, gamma, beta)


def _i16(v):
    return jnp.full((16,), v, jnp.int32)


def _dst_to_local(d16):
    one = _i16(1)
    zero = _i16(0)
    bidx = (jnp.where(d16 >= _i16(N), one, zero)
            + jnp.where(d16 >= _i16(2 * N), one, zero)
            + jnp.where(d16 >= _i16(3 * N), one, zero))
    r = d16 - bidx * _i16(N)
    ent = jnp.logical_and(r >= _i16(NM), r < _i16(NM + NE))
    loc = jnp.where(ent, bidx * _i16(NE) + (r - _i16(NM)), _i16(DUMMY))
    return loc, ent


def _sc_body(edge_hbm, x_hbm, agg_out, deg_out,
             ea, eb, rows_a, rows_b, ones_v, zrow, psrc, ploc,
             acc_sh, deg_sh, sem_ea, sem_eb, sem_a, sem_b):
    c = lax.axis_index("c")
    s = lax.axis_index("s")
    w = c * NSUB + s

    zeros16 = jnp.zeros((16,), jnp.float32)
    ones16 = jnp.ones((16,), jnp.float32)
    for r in range(8):
        for i in range(8):
            zrow[r, pl.ds(16 * i, 16)] = zeros16
    for r in range(CHUNK):
        for i in range(8):
            ones_v[r, pl.ds(16 * i, 16)] = ones16

    base = s * ROWS_PER_TILE
    for k in range(ROWS_PER_TILE // 8):
        pltpu.sync_copy(zrow, acc_sh.at[pl.ds(base + 8 * k, 8)])
        pltpu.sync_copy(zrow, deg_sh.at[pl.ds(base + 8 * k, 8)])

    cbase = w * CHUNKS_PER_W
    ebufs = (ea, eb)
    esems = (sem_ea, sem_eb)
    pltpu.async_copy(edge_hbm.at[:, pl.ds(cbase, EBLK), :], ea, sem_ea)
    cnt_vec = _i16(0)
    for lb in range(NLB):
        buf = ebufs[lb % 2]
        if lb + 1 < NLB:
            pltpu.async_copy(
                edge_hbm.at[:, pl.ds(cbase + (lb + 1) * EBLK, EBLK), :],
                ebufs[(lb + 1) % 2], esems[(lb + 1) % 2])
        pltpu.make_async_copy(
            edge_hbm.at[:, pl.ds(cbase, EBLK), :], buf, esems[lb % 2]).wait()

        def blk_body(kk, cv, buf=buf, lb=lb):
            for i in range(CHUNK // 16):
                s16 = buf[0, kk, pl.ds(16 * i, 16)]
                d16 = buf[1, kk, pl.ds(16 * i, 16)]
                loc, ent = _dst_to_local(d16)
                pos = cv + jnp.cumsum(
                    jnp.where(ent, _i16(1), _i16(0))) - _i16(1)
                plsc.store_scatter(psrc, [pos], s16, mask=ent)
                plsc.store_scatter(ploc, [pos], loc, mask=ent)
                cv = cv + plsc.all_reduce_population_count(ent)
            return cv

        cnt_vec = lax.fori_loop(0, EBLK, blk_body, cnt_vec)

    iota16 = lax.iota(jnp.int32, 16)
    for k in range(CHUNK // 16):
        tail = cnt_vec + iota16 + _i16(16 * k)
        plsc.store_scatter(psrc, [tail], _i16(0))
        plsc.store_scatter(ploc, [tail], _i16(DUMMY))
    cnt = jnp.max(cnt_vec)
    nch = (cnt + (CHUNK - 1)) // CHUNK
    plsc.subcore_barrier()

    rbufs = (rows_a, rows_b)
    rsems = (sem_a, sem_b)

    @pl.when(nch > 0)
    def _():
        pltpu.async_copy(x_hbm.at[psrc.at[pl.ds(0, CHUNK)]], rows_a, sem_a)

    def pair_body(j2, carry):
        for bsel in range(2):
            j = 2 * j2 + bsel
            buf = rbufs[bsel]
            sem = rsems[bsel]

            @pl.when(j < nch)
            def _(j=j, buf=buf, sem=sem, bsel=bsel):
                @pl.when(j + 1 < nch)
                def _():
                    off2 = (j + 1) * CHUNK
                    pltpu.async_copy(
                        x_hbm.at[psrc.at[pl.ds(off2, CHUNK)]],
                        rbufs[1 - bsel], rsems[1 - bsel])

                off = j * CHUNK
                pltpu.sync_copy(ones_v,
                                deg_sh.at[ploc.at[pl.ds(off, CHUNK)]],
                                add=True)
                pltpu.make_async_copy(
                    x_hbm.at[psrc.at[pl.ds(off, CHUNK)]], buf, sem).wait()
                pltpu.sync_copy(buf,
                                acc_sh.at[ploc.at[pl.ds(off, CHUNK)]],
                                add=True)
        return carry

    lax.fori_loop(0, (nch + 1) // 2, pair_body, 0)
    plsc.subcore_barrier()

    pltpu.sync_copy(acc_sh.at[pl.ds(base, ROWS_PER_TILE)],
                    agg_out.at[c, pl.ds(base, ROWS_PER_TILE)])
    pltpu.sync_copy(deg_sh.at[pl.ds(base, ROWS_PER_TILE)],
                    deg_out.at[c, pl.ds(base, ROWS_PER_TILE)])


def _sc_scatter(edges3, x):
    mesh = plsc.VectorSubcoreMesh(core_axis_name="c", subcore_axis_name="s")
    fn = pl.kernel(
        _sc_body,
        out_type=(
            jax.ShapeDtypeStruct((NC, ACC_ROWS, D), jnp.float32),
            jax.ShapeDtypeStruct((NC, ACC_ROWS, D), jnp.float32),
        ),
        mesh=mesh,
        compiler_params=pltpu.CompilerParams(needs_layout_passes=False),
        scratch_types=[
            pltpu.VMEM((2, EBLK, CHUNK), jnp.int32),
            pltpu.VMEM((2, EBLK, CHUNK), jnp.int32),
            pltpu.VMEM((CHUNK, D), jnp.float32),
            pltpu.VMEM((CHUNK, D), jnp.float32),
            pltpu.VMEM((CHUNK, D), jnp.float32),
            pltpu.VMEM((8, D), jnp.float32),
            pltpu.VMEM((PCAP,), jnp.int32),
            pltpu.VMEM((PCAP,), jnp.int32),
            pltpu.VMEM_SHARED((ACC_ROWS, D), jnp.float32),
            pltpu.VMEM_SHARED((ACC_ROWS, D), jnp.float32),
            pltpu.SemaphoreType.DMA,
            pltpu.SemaphoreType.DMA,
            pltpu.SemaphoreType.DMA,
            pltpu.SemaphoreType.DMA,
        ],
    )
    return fn(edges3, x)


def _combine_body(xe_ref, agg_ref, deg_ref, ws_ref, wn_ref, b_ref, o_ref):
    xe = xe_ref[...]
    selfm = jnp.dot(xe, ws_ref[...], preferred_element_type=jnp.float32)
    acc = agg_ref[0] + agg_ref[1]
    aggm = jnp.dot(acc[:B * NE], wn_ref[...], preferred_element_type=jnp.float32)
    deg = deg_ref[0, :, 0:1] + deg_ref[1, :, 0:1]
    d = jnp.maximum(deg[:B * NE], 1.0)
    o_ref[...] = jnp.maximum(selfm + aggm / d + b_ref[...], 0.0)


def _combine(xe, agg, deg, w_self, w_nbr, bias):
    return pl.pallas_call(
        _combine_body,
        out_shape=jax.ShapeDtypeStruct((B * NE, D), jnp.float32),
    )(xe, agg, deg, w_self, w_nbr, bias)


def kernel(mention_hidden_state, entity_hidden_state, sent_hidden_state,
           virtual_hidden_state, edge_index, type_emb, ln_gamma, ln_beta,
           W_self, W_nbr, b):
    feats = jnp.concatenate(
        [mention_hidden_state, entity_hidden_state,
         sent_hidden_state, virtual_hidden_state], axis=1).reshape(BN, H)
    tnode = jnp.concatenate([
        jnp.broadcast_to(type_emb[0], (NM, TE)),
        jnp.broadcast_to(type_emb[1], (NE, TE)),
        jnp.broadcast_to(type_emb[2], (NS, TE)),
        jnp.broadcast_to(type_emb[3], (NV, TE)),
    ], axis=0)
    trows = jnp.broadcast_to(tnode[None], (B, N, TE)).reshape(BN, TE)

    x = _ln_concat(feats, trows, ln_gamma.reshape(1, D), ln_beta.reshape(1, D))

    edges3 = jnp.pad(edge_index, ((0, 0), (0, EP - EDGES))).reshape(2, NCH, CHUNK)
    agg, deg = _sc_scatter(edges3, x)

    xe = x.reshape(B, N, D)[:, NM:NM + NE].reshape(B * NE, D)
    out = _combine(xe, agg, deg, W_self, W_nbr, b.reshape(1, D))
    return out.reshape(B, NE, D)


# speedup vs baseline: 11.3319x; 1.0214x over previous
import functools

import jax
import jax.numpy as jnp
from jax import lax
from jax.experimental import pallas as pl
from jax.experimental.pallas import tpu as pltpu, tpu_sc as plsc

B, NM, NE, NS, NV = 4, 1200, 600, 500, 200
H, TE, D = 96, 32, 128
N = NM + NE + NS + NV
BN = B * N
EDGES = 320000

NC, NSUB = 2, 16
NW = NC * NSUB
CHUNK = 128
EBLK = 8
NLB = 10
CHUNKS_PER_W = EBLK * NLB
EP = NW * CHUNKS_PER_W * CHUNK
NCH = EP // CHUNK
EDGES_PER_W = CHUNKS_PER_W * CHUNK
PCAP = EDGES_PER_W + CHUNK

ACC_ROWS = 2432
DUMMY = 2400
ROWS_PER_TILE = ACC_ROWS // NSUB


def _ln_body(f_ref, t_ref, g_ref, b_ref, x_ref):
    x = jnp.concatenate([f_ref[...], t_ref[...]], axis=1)
    mu = jnp.mean(x, axis=1, keepdims=True)
    var = jnp.mean((x - mu) * (x - mu), axis=1, keepdims=True)
    xn = (x - mu) * lax.rsqrt(var + 1e-5)
    x_ref[...] = xn * g_ref[...] + b_ref[...]


def _ln_concat(feats, trows, gamma, beta):
    blk = 1000
    grid = BN // blk
    return pl.pallas_call(
        _ln_body,
        grid=(grid,),
        in_specs=[
            pl.BlockSpec((blk, H), lambda i: (i, 0)),
            pl.BlockSpec((blk, TE), lambda i: (i, 0)),
            pl.BlockSpec((1, D), lambda i: (0, 0)),
            pl.BlockSpec((1, D), lambda i: (0, 0)),
        ],
        out_specs=pl.BlockSpec((blk, D), lambda i: (i, 0)),
        out_shape=jax.ShapeDtypeStruct((BN, D), jnp.float32),
    )(feats, trows, gamma, beta)


def _i16(v):
    return jnp.full((16,), v, jnp.int32)


def _dst_to_local(d16):
    one = _i16(1)
    zero = _i16(0)
    bidx = (jnp.where(d16 >= _i16(N), one, zero)
            + jnp.where(d16 >= _i16(2 * N), one, zero)
            + jnp.where(d16 >= _i16(3 * N), one, zero))
    r = d16 - bidx * _i16(N)
    ent = jnp.logical_and(r >= _i16(NM), r < _i16(NM + NE))
    loc = jnp.where(ent, bidx * _i16(NE) + (r - _i16(NM)), _i16(DUMMY))
    return loc, ent


def _sc_body(edge_hbm, x_hbm, agg_out, deg_out,
             ea, eb, rows_a, rows_b, hist, zrow, psrc, ploc,
             acc_sh, sem_ea, sem_eb, sem_a, sem_b):
    c = lax.axis_index("c")
    s = lax.axis_index("s")
    w = c * NSUB + s

    zeros16 = jnp.zeros((16,), jnp.float32)
    ones16 = jnp.ones((16,), jnp.float32)
    for r in range(8):
        for i in range(8):
            zrow[r, pl.ds(16 * i, 16)] = zeros16
    for i in range(ACC_ROWS // 16):
        hist[pl.ds(16 * i, 16)] = zeros16

    base = s * ROWS_PER_TILE
    for k in range(ROWS_PER_TILE // 8):
        pltpu.sync_copy(zrow, acc_sh.at[pl.ds(base + 8 * k, 8)])

    cbase = w * CHUNKS_PER_W
    ebufs = (ea, eb)
    esems = (sem_ea, sem_eb)
    pltpu.async_copy(edge_hbm.at[:, pl.ds(cbase, EBLK), :], ea, sem_ea)
    cnt_vec = _i16(0)
    for lb in range(NLB):
        buf = ebufs[lb % 2]
        if lb + 1 < NLB:
            pltpu.async_copy(
                edge_hbm.at[:, pl.ds(cbase + (lb + 1) * EBLK, EBLK), :],
                ebufs[(lb + 1) % 2], esems[(lb + 1) % 2])
        pltpu.make_async_copy(
            edge_hbm.at[:, pl.ds(cbase, EBLK), :], buf, esems[lb % 2]).wait()

        def blk_body(kk, cv, buf=buf, lb=lb):
            for i in range(CHUNK // 16):
                s16 = buf[0, kk, pl.ds(16 * i, 16)]
                d16 = buf[1, kk, pl.ds(16 * i, 16)]
                loc, ent = _dst_to_local(d16)
                pos = cv + jnp.cumsum(
                    jnp.where(ent, _i16(1), _i16(0))) - _i16(1)
                plsc.store_scatter(psrc, [pos], s16, mask=ent)
                plsc.store_scatter(ploc, [pos], loc, mask=ent)
                plsc.addupdate_scatter(hist, [loc], ones16)
                cv = cv + plsc.all_reduce_population_count(ent)
            return cv

        cnt_vec = lax.fori_loop(0, EBLK, blk_body, cnt_vec)

    iota16 = lax.iota(jnp.int32, 16)
    for k in range(CHUNK // 16):
        tail = cnt_vec + iota16 + _i16(16 * k)
        plsc.store_scatter(psrc, [tail], _i16(0))
        plsc.store_scatter(ploc, [tail], _i16(DUMMY))
    cnt = jnp.max(cnt_vec)
    nch = (cnt + (CHUNK - 1)) // CHUNK
    plsc.subcore_barrier()

    rbufs = (rows_a, rows_b)
    rsems = (sem_a, sem_b)

    @pl.when(nch > 0)
    def _():
        pltpu.async_copy(x_hbm.at[psrc.at[pl.ds(0, CHUNK)]], rows_a, sem_a)

    def pair_body(j2, carry):
        for bsel in range(2):
            j = 2 * j2 + bsel
            buf = rbufs[bsel]
            sem = rsems[bsel]

            @pl.when(j < nch)
            def _(j=j, buf=buf, sem=sem, bsel=bsel):
                @pl.when(j + 1 < nch)
                def _():
                    off2 = (j + 1) * CHUNK
                    pltpu.async_copy(
                        x_hbm.at[psrc.at[pl.ds(off2, CHUNK)]],
                        rbufs[1 - bsel], rsems[1 - bsel])

                off = j * CHUNK
                pltpu.make_async_copy(
                    x_hbm.at[psrc.at[pl.ds(off, CHUNK)]], buf, sem).wait()
                pltpu.sync_copy(buf,
                                acc_sh.at[ploc.at[pl.ds(off, CHUNK)]],
                                add=True)
        return carry

    lax.fori_loop(0, (nch + 1) // 2, pair_body, 0)
    plsc.subcore_barrier()

    pltpu.sync_copy(acc_sh.at[pl.ds(base, ROWS_PER_TILE)],
                    agg_out.at[c, pl.ds(base, ROWS_PER_TILE)])
    pltpu.sync_copy(hist, deg_out.at[c, s])


def _sc_scatter(edges3, x):
    mesh = plsc.VectorSubcoreMesh(core_axis_name="c", subcore_axis_name="s")
    fn = pl.kernel(
        _sc_body,
        out_type=(
            jax.ShapeDtypeStruct((NC, ACC_ROWS, D), jnp.float32),
            jax.ShapeDtypeStruct((NC, NSUB, ACC_ROWS), jnp.float32),
        ),
        mesh=mesh,
        compiler_params=pltpu.CompilerParams(needs_layout_passes=False),
        scratch_types=[
            pltpu.VMEM((2, EBLK, CHUNK), jnp.int32),
            pltpu.VMEM((2, EBLK, CHUNK), jnp.int32),
            pltpu.VMEM((CHUNK, D), jnp.float32),
            pltpu.VMEM((CHUNK, D), jnp.float32),
            pltpu.VMEM((ACC_ROWS,), jnp.float32),
            pltpu.VMEM((8, D), jnp.float32),
            pltpu.VMEM((PCAP,), jnp.int32),
            pltpu.VMEM((PCAP,), jnp.int32),
            pltpu.VMEM_SHARED((ACC_ROWS, D), jnp.float32),
            pltpu.SemaphoreType.DMA,
            pltpu.SemaphoreType.DMA,
            pltpu.SemaphoreType.DMA,
            pltpu.SemaphoreType.DMA,
        ],
    )
    return fn(edges3, x)


def _combine_body(xe_ref, agg_ref, dcol_ref, ws_ref, wn_ref, b_ref, o_ref):
    xe = xe_ref[...]
    selfm = jnp.dot(xe, ws_ref[...], preferred_element_type=jnp.float32)
    acc = agg_ref[0] + agg_ref[1]
    aggm = jnp.dot(acc[:B * NE], wn_ref[...], preferred_element_type=jnp.float32)
    deg = jnp.sum(dcol_ref[...], axis=1, keepdims=True)
    d = jnp.maximum(deg[:B * NE], 1.0)
    o_ref[...] = jnp.maximum(selfm + aggm / d + b_ref[...], 0.0)


def _combine(xe, agg, dcol, w_self, w_nbr, bias):
    return pl.pallas_call(
        _combine_body,
        out_shape=jax.ShapeDtypeStruct((B * NE, D), jnp.float32),
    )(xe, agg, dcol, w_self, w_nbr, bias)


def kernel(mention_hidden_state, entity_hidden_state, sent_hidden_state,
           virtual_hidden_state, edge_index, type_emb, ln_gamma, ln_beta,
           W_self, W_nbr, b):
    feats = jnp.concatenate(
        [mention_hidden_state, entity_hidden_state,
         sent_hidden_state, virtual_hidden_state], axis=1).reshape(BN, H)
    tnode = jnp.concatenate([
        jnp.broadcast_to(type_emb[0], (NM, TE)),
        jnp.broadcast_to(type_emb[1], (NE, TE)),
        jnp.broadcast_to(type_emb[2], (NS, TE)),
        jnp.broadcast_to(type_emb[3], (NV, TE)),
    ], axis=0)
    trows = jnp.broadcast_to(tnode[None], (B, N, TE)).reshape(BN, TE)

    x = _ln_concat(feats, trows, ln_gamma.reshape(1, D), ln_beta.reshape(1, D))

    edges3 = jnp.pad(edge_index, ((0, 0), (0, EP - EDGES))).reshape(2, NCH, CHUNK)
    agg, deg = _sc_scatter(edges3, x)

    xe = x.reshape(B, N, D)[:, NM:NM + NE].reshape(B * NE, D)
    dcol = deg.reshape(NW, ACC_ROWS).T
    out = _combine(xe, agg, dcol, W_self, W_nbr, b.reshape(1, D))
    return out.reshape(B, NE, D)


# speedup vs baseline: 11.3566x; 1.0022x over previous
import functools

import jax
import jax.numpy as jnp
from jax import lax
from jax.experimental import pallas as pl
from jax.experimental.pallas import tpu as pltpu, tpu_sc as plsc

B, NM, NE, NS, NV = 4, 1200, 600, 500, 200
H, TE, D = 96, 32, 128
N = NM + NE + NS + NV
BN = B * N
EDGES = 320000

NC, NSUB = 2, 16
NW = NC * NSUB
CHUNK = 128
EBLK = 8
NLB = 10
CHUNKS_PER_W = EBLK * NLB
EP = NW * CHUNKS_PER_W * CHUNK
NCH = EP // CHUNK
EDGES_PER_W = CHUNKS_PER_W * CHUNK
PCAP = EDGES_PER_W + CHUNK

ACC_ROWS = 2432
DUMMY = 2400
ROWS_PER_TILE = ACC_ROWS // NSUB


def _ln_body(f_ref, t_ref, g_ref, b_ref, x_ref):
    x = jnp.concatenate([f_ref[...], t_ref[...]], axis=1)
    mu = jnp.mean(x, axis=1, keepdims=True)
    var = jnp.mean((x - mu) * (x - mu), axis=1, keepdims=True)
    xn = (x - mu) * lax.rsqrt(var + 1e-5)
    x_ref[...] = xn * g_ref[...] + b_ref[...]


def _ln_concat(feats, trows, gamma, beta):
    blk = 1000
    grid = BN // blk
    return pl.pallas_call(
        _ln_body,
        grid=(grid,),
        in_specs=[
            pl.BlockSpec((blk, H), lambda i: (i, 0)),
            pl.BlockSpec((blk, TE), lambda i: (i, 0)),
            pl.BlockSpec((1, D), lambda i: (0, 0)),
            pl.BlockSpec((1, D), lambda i: (0, 0)),
        ],
        out_specs=pl.BlockSpec((blk, D), lambda i: (i, 0)),
        out_shape=jax.ShapeDtypeStruct((BN, D), jnp.float32),
    )(feats, trows, gamma, beta)


def _i16(v):
    return jnp.full((16,), v, jnp.int32)


def _dst_to_local(d16):
    one = _i16(1)
    zero = _i16(0)
    bidx = (jnp.where(d16 >= _i16(N), one, zero)
            + jnp.where(d16 >= _i16(2 * N), one, zero)
            + jnp.where(d16 >= _i16(3 * N), one, zero))
    r = d16 - bidx * _i16(N)
    ent = jnp.logical_and(r >= _i16(NM), r < _i16(NM + NE))
    loc = jnp.where(ent, bidx * _i16(NE) + (r - _i16(NM)), _i16(DUMMY))
    return loc, ent


def _sc_body(edge_hbm, x_hbm, agg_out, deg_out,
             ea, eb, rows_a, rows_b, rows_c, hist, zrow, psrc, ploc,
             acc_sh, sem_ea, sem_eb, sem_a, sem_b, sem_c,
             sem_sa, sem_sb, sem_sc):
    c = lax.axis_index("c")
    s = lax.axis_index("s")
    w = c * NSUB + s

    zeros16 = jnp.zeros((16,), jnp.float32)
    ones16 = jnp.ones((16,), jnp.float32)
    for r in range(8):
        for i in range(8):
            zrow[r, pl.ds(16 * i, 16)] = zeros16
    for i in range(ACC_ROWS // 16):
        hist[pl.ds(16 * i, 16)] = zeros16

    base = s * ROWS_PER_TILE
    for k in range(ROWS_PER_TILE // 8):
        pltpu.sync_copy(zrow, acc_sh.at[pl.ds(base + 8 * k, 8)])

    cbase = w * CHUNKS_PER_W
    ebufs = (ea, eb)
    esems = (sem_ea, sem_eb)
    pltpu.async_copy(edge_hbm.at[:, pl.ds(cbase, EBLK), :], ea, sem_ea)
    cnt_vec = _i16(0)
    for lb in range(NLB):
        buf = ebufs[lb % 2]
        if lb + 1 < NLB:
            pltpu.async_copy(
                edge_hbm.at[:, pl.ds(cbase + (lb + 1) * EBLK, EBLK), :],
                ebufs[(lb + 1) % 2], esems[(lb + 1) % 2])
        pltpu.make_async_copy(
            edge_hbm.at[:, pl.ds(cbase, EBLK), :], buf, esems[lb % 2]).wait()

        def blk_body(kk, cv, buf=buf, lb=lb):
            for i in range(CHUNK // 16):
                s16 = buf[0, kk, pl.ds(16 * i, 16)]
                d16 = buf[1, kk, pl.ds(16 * i, 16)]
                loc, ent = _dst_to_local(d16)
                pos = cv + jnp.cumsum(
                    jnp.where(ent, _i16(1), _i16(0))) - _i16(1)
                plsc.store_scatter(psrc, [pos], s16, mask=ent)
                plsc.store_scatter(ploc, [pos], loc, mask=ent)
                plsc.addupdate_scatter(hist, [loc], ones16)
                cv = cv + plsc.all_reduce_population_count(ent)
            return cv

        cnt_vec = lax.fori_loop(0, EBLK, blk_body, cnt_vec)

    iota16 = lax.iota(jnp.int32, 16)
    for k in range(CHUNK // 16):
        tail = cnt_vec + iota16 + _i16(16 * k)
        plsc.store_scatter(psrc, [tail], _i16(0))
        plsc.store_scatter(ploc, [tail], _i16(DUMMY))
    cnt = jnp.max(cnt_vec)
    nch = (cnt + (CHUNK - 1)) // CHUNK
    plsc.subcore_barrier()

    rbufs = (rows_a, rows_b, rows_c)
    gsems = (sem_a, sem_b, sem_c)
    ssems = (sem_sa, sem_sb, sem_sc)

    def _gather(j, bsel):
        pltpu.async_copy(x_hbm.at[psrc.at[pl.ds(j * CHUNK, CHUNK)]],
                         rbufs[bsel], gsems[bsel])

    def _scatter_ref(j):
        return acc_sh.at[ploc.at[pl.ds(j * CHUNK, CHUNK)]]

    for k in range(3):
        @pl.when(k < nch)
        def _(k=k):
            _gather(k, k)

    def triple_body(j2, carry):
        for bsel in range(3):
            j = 3 * j2 + bsel
            prev = (bsel + 2) % 3

            @pl.when(j < nch)
            def _(j=j, bsel=bsel):
                pltpu.make_async_copy(
                    x_hbm.at[psrc.at[pl.ds(j * CHUNK, CHUNK)]],
                    rbufs[bsel], gsems[bsel]).wait()
                pltpu.async_copy(rbufs[bsel], _scatter_ref(j), ssems[bsel],
                                 add=True)

            @pl.when(jnp.logical_and(j >= 1, j - 1 < nch))
            def _(j=j, prev=prev):
                pltpu.make_async_copy(rbufs[prev], _scatter_ref(j - 1),
                                      ssems[prev]).wait()

                @pl.when(j + 2 < nch)
                def _():
                    _gather(j + 2, prev)
        return carry

    lax.fori_loop(0, (nch + 3) // 3, triple_body, 0)
    plsc.subcore_barrier()

    pltpu.sync_copy(acc_sh.at[pl.ds(base, ROWS_PER_TILE)],
                    agg_out.at[c, pl.ds(base, ROWS_PER_TILE)])
    pltpu.sync_copy(hist, deg_out.at[c, s])


def _sc_scatter(edges3, x):
    mesh = plsc.VectorSubcoreMesh(core_axis_name="c", subcore_axis_name="s")
    fn = pl.kernel(
        _sc_body,
        out_type=(
            jax.ShapeDtypeStruct((NC, ACC_ROWS, D), jnp.float32),
            jax.ShapeDtypeStruct((NC, NSUB, ACC_ROWS), jnp.float32),
        ),
        mesh=mesh,
        compiler_params=pltpu.CompilerParams(needs_layout_passes=False),
        scratch_types=[
            pltpu.VMEM((2, EBLK, CHUNK), jnp.int32),
            pltpu.VMEM((2, EBLK, CHUNK), jnp.int32),
            pltpu.VMEM((CHUNK, D), jnp.float32),
            pltpu.VMEM((CHUNK, D), jnp.float32),
            pltpu.VMEM((CHUNK, D), jnp.float32),
            pltpu.VMEM((ACC_ROWS,), jnp.float32),
            pltpu.VMEM((8, D), jnp.float32),
            pltpu.VMEM((PCAP,), jnp.int32),
            pltpu.VMEM((PCAP,), jnp.int32),
            pltpu.VMEM_SHARED((ACC_ROWS, D), jnp.float32),
            pltpu.SemaphoreType.DMA,
            pltpu.SemaphoreType.DMA,
            pltpu.SemaphoreType.DMA,
            pltpu.SemaphoreType.DMA,
            pltpu.SemaphoreType.DMA,
            pltpu.SemaphoreType.DMA,
            pltpu.SemaphoreType.DMA,
            pltpu.SemaphoreType.DMA,
        ],
    )
    return fn(edges3, x)


def _combine_body(xe_ref, agg_ref, dcol_ref, ws_ref, wn_ref, b_ref, o_ref):
    xe = xe_ref[...]
    selfm = jnp.dot(xe, ws_ref[...], preferred_element_type=jnp.float32)
    acc = agg_ref[0] + agg_ref[1]
    aggm = jnp.dot(acc[:B * NE], wn_ref[...], preferred_element_type=jnp.float32)
    deg = jnp.sum(dcol_ref[...], axis=1, keepdims=True)
    d = jnp.maximum(deg[:B * NE], 1.0)
    o_ref[...] = jnp.maximum(selfm + aggm / d + b_ref[...], 0.0)


def _combine(xe, agg, dcol, w_self, w_nbr, bias):
    return pl.pallas_call(
        _combine_body,
        out_shape=jax.ShapeDtypeStruct((B * NE, D), jnp.float32),
    )(xe, agg, dcol, w_self, w_nbr, bias)


def kernel(mention_hidden_state, entity_hidden_state, sent_hidden_state,
           virtual_hidden_state, edge_index, type_emb, ln_gamma, ln_beta,
           W_self, W_nbr, b):
    feats = jnp.concatenate(
        [mention_hidden_state, entity_hidden_state,
         sent_hidden_state, virtual_hidden_state], axis=1).reshape(BN, H)
    tnode = jnp.concatenate([
        jnp.broadcast_to(type_emb[0], (NM, TE)),
        jnp.broadcast_to(type_emb[1], (NE, TE)),
        jnp.broadcast_to(type_emb[2], (NS, TE)),
        jnp.broadcast_to(type_emb[3], (NV, TE)),
    ], axis=0)
    trows = jnp.broadcast_to(tnode[None], (B, N, TE)).reshape(BN, TE)

    x = _ln_concat(feats, trows, ln_gamma.reshape(1, D), ln_beta.reshape(1, D))

    edges3 = jnp.pad(edge_index, ((0, 0), (0, EP - EDGES))).reshape(2, NCH, CHUNK)
    agg, deg = _sc_scatter(edges3, x)

    xe = x.reshape(B, N, D)[:, NM:NM + NE].reshape(B * NE, D)
    dcol = deg.reshape(NW, ACC_ROWS).T
    out = _combine(xe, agg, dcol, W_self, W_nbr, b.reshape(1, D))
    return out.reshape(B, NE, D)
